# baseline (speedup 1.0000x reference)
import functools

import jax
import jax.numpy as jnp
from jax.experimental import pallas as pl

ATTR_W = 1.0
REPL_W = 1.0
MARGIN_W = 5.0
THR = 0.5
MARG = 0.2

NUM_INST = 64
ROW_TILE = 128
LOG2E = 1.4426950408889634


def _oc_kernel(beta_ref, emb_ref, sid_ref, iscp_ref, iscp_col_ref, out_ref):
    n = emb_ref.shape[1]
    d = emb_ref.shape[2]
    nb = pl.num_programs(0)
    i = pl.program_id(0)
    x_col = beta_ref[0]
    emb = emb_ref[0]
    sid = sid_ref[pl.ds(i, 1), :]
    iscp = iscp_ref[0]
    pos_col = (iscp_col_ref[0] == 1).astype(jnp.float32)

    pos = (iscp == 1).astype(jnp.float32)
    neg = 1.0 - pos
    pos_count = jnp.sum(pos)
    neg_count = jnp.sum(neg)
    valid = jnp.where((pos_count >= 1.0) & (neg_count >= 1.0), 1.0, 0.0)

    neg_col = 1.0 - pos_col
    pw = neg_count / (pos_count + 1e-6)
    sp_neg = jax.nn.softplus(-x_col)
    sp_pos = jax.nn.softplus(x_col)
    bce = jnp.sum(pw * pos_col * sp_neg + neg_col * sp_pos) / n
    prob = jax.nn.sigmoid(x_col)
    pos_m = jnp.sum(jax.nn.relu(THR + MARG - prob) * pos_col) / jnp.maximum(pos_count, 1.0)
    neg_m = jnp.sum(jax.nn.relu(prob - (THR - MARG)) * neg_col) / jnp.maximum(neg_count, 1.0)
    beta_loss = bce + MARGIN_W * (pos_m + neg_m)

    n2c = jnp.sum(emb * emb, axis=1, keepdims=True)
    inst_iota = jax.lax.broadcasted_iota(jnp.int32, (NUM_INST, n), 0)
    m = (sid == inst_iota).astype(jnp.float32)
    ones_col = jnp.ones((n, 1), jnp.float32)
    emb_aug = jnp.concatenate([emb, n2c, ones_col], axis=1)
    s_aug = jnp.dot(m, emb_aug, preferred_element_type=jnp.float32)
    s1 = s_aug[:, :d]
    s2 = s_aug[:, d]
    cnt = s_aug[:, d + 1]

    col_iota = jax.lax.broadcasted_iota(jnp.int32, (NUM_INST, n), 1)
    cp_inst = (m > 0.0) & (pos > 0.0)
    first = jnp.min(jnp.where(cp_inst, col_iota, n), axis=1)
    has_cp = first < n
    first_c = jnp.where(has_cp, first, 0)
    pick = (col_iota == first_c[:, None]).astype(jnp.float32)
    cp_ref = jnp.dot(pick, emb, preferred_element_type=jnp.float32)
    cp_n2 = jnp.sum(cp_ref * cp_ref, axis=1)

    mean_d2 = (s2 - 2.0 * jnp.sum(cp_ref * s1, axis=1) + cnt * cp_n2) / jnp.maximum(cnt, 1.0)
    attraction = jnp.sum(jnp.where(has_cp, mean_d2, 0.0)) * ATTR_W

    c_shift = 0.5 * jnp.max(n2c)
    es = emb * jnp.float32((2.0 * LOG2E) ** 0.5)
    nc = jnp.where(pos_col > 0.0, LOG2E * (n2c + c_shift), jnp.float32(1.0e9))
    lu = jnp.where(pos_col > 0.0, LOG2E * (c_shift - n2c), jnp.float32(-1.0e9))
    row_aug = jnp.concatenate([es, lu, ones_col], axis=1)
    col_aug = jnp.concatenate([es, ones_col, -nc], axis=1)

    ri = jax.lax.broadcasted_iota(jnp.int32, (ROW_TILE, ROW_TILE), 0)
    ci = jax.lax.broadcasted_iota(jnp.int32, (ROW_TILE, ROW_TILE), 1)
    diag_mask = (ci > ri).astype(jnp.float32)

    upper = jnp.float32(0.0)
    for t in range(n // ROW_TILE):
        c0 = t * ROW_TILE
        et = row_aug[c0:c0 + ROW_TILE, :]
        ec = col_aug[c0:, :]
        ex = jax.lax.dot_general(et, ec, (((1,), (1,)), ((), ())),
                                 preferred_element_type=jnp.float32)
        w = jnp.exp2(ex)
        upper = upper + jnp.sum(w[:, :ROW_TILE] * diag_mask)
        if c0 + ROW_TILE < n:
            upper = upper + jnp.sum(w[:, ROW_TILE:])

    pair_sum = pos_count + 2.0 * upper
    repulsion = jnp.where(
        pos_count > 1.0,
        pair_sum / jnp.maximum(pos_count * pos_count, 1.0),
        0.0,
    ) * REPL_W

    lane = jax.lax.broadcasted_iota(jnp.int32, (1, 128), 1)
    total_b = valid * (beta_loss + attraction + repulsion)
    outvec = (
        jnp.where(lane == 0, total_b, 0.0)
        + jnp.where(lane == 1, valid * beta_loss, 0.0)
        + jnp.where(lane == 2, valid * attraction, 0.0)
        + jnp.where(lane == 3, valid * repulsion, 0.0)
        + jnp.where(lane == 4, valid, 0.0)
    )

    i = pl.program_id(0)

    @pl.when(i == 0)
    def _():
        out_ref[0] = outvec

    @pl.when(i > 0)
    def _():
        acc = out_ref[0] + outvec

        @pl.when(i == nb - 1)
        def _():
            count = jnp.sum(jnp.where(lane == 4, acc, 0.0))
            denom = jnp.maximum(count, 1.0)
            scale = jnp.where(count > 0.0, 1.0 / denom, 0.0)
            out_ref[0] = acc * scale

        @pl.when(i < nb - 1)
        def _():
            out_ref[0] = acc


@functools.partial(jax.jit, static_argnames=())
def kernel(beta, embed, slice_id, is_cp):
    b, n, d = embed.shape
    sid = slice_id.astype(jnp.int32)
    iscp = jnp.reshape(is_cp.astype(jnp.int32), (b, 1, n))
    iscp_col = jnp.reshape(is_cp.astype(jnp.int32), (b, n, 1))

    acc = pl.pallas_call(
        _oc_kernel,
        grid=(b,),
        in_specs=[
            pl.BlockSpec((1, n, 1), lambda i: (i, 0, 0)),
            pl.BlockSpec((1, n, d), lambda i: (i, 0, 0)),
            pl.BlockSpec((b, n), lambda i: (0, 0)),
            pl.BlockSpec((1, 1, n), lambda i: (i, 0, 0)),
            pl.BlockSpec((1, n, 1), lambda i: (i, 0, 0)),
        ],
        out_specs=pl.BlockSpec((1, 1, 128), lambda i: (0, 0, 0)),
        out_shape=jax.ShapeDtypeStruct((1, 1, 128), jnp.float32),
    )(beta, embed, sid, iscp, iscp_col)

    v = acc[0, 0]
    return (v[0], v[1], v[2], v[3])


# speedup vs baseline: 1.5107x; 1.5107x over previous
import functools

import jax
import jax.numpy as jnp
from jax.experimental import pallas as pl

ATTR_W = 1.0
REPL_W = 1.0
MARGIN_W = 5.0
THR = 0.5
MARG = 0.2

NUM_INST = 64
ROW_TILE = 128
LOG2E = 1.4426950408889634


def _oc_kernel(beta_ref, emb_ref, sid_ref, iscp_ref, iscp_col_ref, out_ref):
    n = emb_ref.shape[1]
    d = emb_ref.shape[2]
    nb = pl.num_programs(0)
    x = beta_ref[0]
    emb = emb_ref[0]
    sid = sid_ref[0]
    iscp = iscp_ref[0]
    pos_col = (iscp_col_ref[0] == 1).astype(jnp.float32)

    pos = (iscp == 1).astype(jnp.float32)
    neg = 1.0 - pos
    pos_count = jnp.sum(pos)
    neg_count = jnp.sum(neg)
    valid = jnp.where((pos_count >= 1.0) & (neg_count >= 1.0), 1.0, 0.0)

    pw = neg_count / (pos_count + 1e-6)
    sp_neg = jax.nn.softplus(-x)
    sp_pos = jax.nn.softplus(x)
    bce = jnp.sum(pw * pos * sp_neg + neg * sp_pos) / n
    prob = jax.nn.sigmoid(x)
    pos_m = jnp.sum(jax.nn.relu(THR + MARG - prob) * pos) / jnp.maximum(pos_count, 1.0)
    neg_m = jnp.sum(jax.nn.relu(prob - (THR - MARG)) * neg) / jnp.maximum(neg_count, 1.0)
    beta_loss = bce + MARGIN_W * (pos_m + neg_m)

    n2c = jnp.sum(emb * emb, axis=1, keepdims=True)
    inst_iota = jax.lax.broadcasted_iota(jnp.int32, (NUM_INST, n), 0)
    m = (sid == inst_iota).astype(jnp.float32)
    ones_col = jnp.ones((n, 1), jnp.float32)
    emb_aug = jnp.concatenate([emb, n2c, ones_col], axis=1)
    s_aug = jnp.dot(m, emb_aug, preferred_element_type=jnp.float32)
    s1 = s_aug[:, :d]
    s2 = s_aug[:, d]
    cnt = s_aug[:, d + 1]

    col_iota = jax.lax.broadcasted_iota(jnp.int32, (NUM_INST, n), 1)
    cp_inst = (m > 0.0) & (pos > 0.0)
    first = jnp.min(jnp.where(cp_inst, col_iota, n), axis=1)
    has_cp = first < n
    first_c = jnp.where(has_cp, first, 0)
    pick = (col_iota == first_c[:, None]).astype(jnp.float32)
    cp_ref = jnp.dot(pick, emb, preferred_element_type=jnp.float32)
    cp_n2 = jnp.sum(cp_ref * cp_ref, axis=1)

    mean_d2 = (s2 - 2.0 * jnp.sum(cp_ref * s1, axis=1) + cnt * cp_n2) / jnp.maximum(cnt, 1.0)
    attraction = jnp.sum(jnp.where(has_cp, mean_d2, 0.0)) * ATTR_W

    c_shift = 0.5 * jnp.max(n2c)
    es = emb * jnp.float32((2.0 * LOG2E) ** 0.5)
    nc = jnp.where(pos_col > 0.0, LOG2E * (n2c + c_shift), jnp.float32(1.0e9))
    lu = jnp.where(pos_col > 0.0, LOG2E * (c_shift - n2c), jnp.float32(-1.0e9))
    row_aug = jnp.concatenate([es, lu, ones_col], axis=1)
    col_aug = jnp.concatenate([es, ones_col, -nc], axis=1)

    ri = jax.lax.broadcasted_iota(jnp.int32, (ROW_TILE, ROW_TILE), 0)
    ci = jax.lax.broadcasted_iota(jnp.int32, (ROW_TILE, ROW_TILE), 1)
    diag_mask = (ci > ri).astype(jnp.float32)

    upper = jnp.float32(0.0)
    for t in range(n // ROW_TILE):
        c0 = t * ROW_TILE
        et = row_aug[c0:c0 + ROW_TILE, :]
        ec = col_aug[c0:, :]
        ex = jax.lax.dot_general(et, ec, (((1,), (1,)), ((), ())),
                                 preferred_element_type=jnp.float32)
        w = jnp.exp2(ex)
        upper = upper + jnp.sum(w[:, :ROW_TILE] * diag_mask)
        if c0 + ROW_TILE < n:
            upper = upper + jnp.sum(w[:, ROW_TILE:])

    pair_sum = pos_count + 2.0 * upper
    repulsion = jnp.where(
        pos_count > 1.0,
        pair_sum / jnp.maximum(pos_count * pos_count, 1.0),
        0.0,
    ) * REPL_W

    lane = jax.lax.broadcasted_iota(jnp.int32, (1, 128), 1)
    total_b = valid * (beta_loss + attraction + repulsion)
    outvec = (
        jnp.where(lane == 0, total_b, 0.0)
        + jnp.where(lane == 1, valid * beta_loss, 0.0)
        + jnp.where(lane == 2, valid * attraction, 0.0)
        + jnp.where(lane == 3, valid * repulsion, 0.0)
        + jnp.where(lane == 4, valid, 0.0)
    )

    i = pl.program_id(0)

    @pl.when(i == 0)
    def _():
        out_ref[0] = outvec

    @pl.when(i > 0)
    def _():
        acc = out_ref[0] + outvec

        @pl.when(i == nb - 1)
        def _():
            count = jnp.sum(jnp.where(lane == 4, acc, 0.0))
            denom = jnp.maximum(count, 1.0)
            scale = jnp.where(count > 0.0, 1.0 / denom, 0.0)
            out_ref[0] = acc * scale

        @pl.when(i < nb - 1)
        def _():
            out_ref[0] = acc


@functools.partial(jax.jit, static_argnames=())
def kernel(beta, embed, slice_id, is_cp):
    b, n, d = embed.shape
    beta_s = jnp.reshape(beta, (b, 1, n))
    sid = jnp.reshape(slice_id.astype(jnp.int32), (b, 1, n))
    iscp = jnp.reshape(is_cp.astype(jnp.int32), (b, 1, n))
    iscp_col = jnp.reshape(is_cp.astype(jnp.int32), (b, n, 1))

    acc = pl.pallas_call(
        _oc_kernel,
        grid=(b,),
        in_specs=[
            pl.BlockSpec((1, 1, n), lambda i: (i, 0, 0)),
            pl.BlockSpec((1, n, d), lambda i: (i, 0, 0)),
            pl.BlockSpec((1, 1, n), lambda i: (i, 0, 0)),
            pl.BlockSpec((1, 1, n), lambda i: (i, 0, 0)),
            pl.BlockSpec((1, n, 1), lambda i: (i, 0, 0)),
        ],
        out_specs=pl.BlockSpec((1, 1, 128), lambda i: (0, 0, 0)),
        out_shape=jax.ShapeDtypeStruct((1, 1, 128), jnp.float32),
    )(beta_s, embed, sid, iscp, iscp_col)

    v = acc[0, 0]
    return (v[0], v[1], v[2], v[3])
